# baseline (speedup 1.0000x reference)
import functools

import jax
import jax.numpy as jnp
from jax import lax
from jax.experimental import pallas as pl
from jax.experimental.pallas import tpu as pltpu
from jax.experimental.pallas import tpu_sc as plsc

D_MODEL = 64
NUM_CORES = 2
NUM_SUBCORES = 16
NW = NUM_CORES * NUM_SUBCORES
CHUNK = 256


def _make_embed(n_tok: int):
    b_per_w = n_tok // NW
    n_chunks = b_per_w // CHUNK

    mesh = plsc.VectorSubcoreMesh(core_axis_name="c", subcore_axis_name="s")

    NBUF = 4
    PF = 3

    @functools.partial(
        pl.kernel,
        mesh=mesh,
        out_type=jax.ShapeDtypeStruct((n_tok, D_MODEL), jnp.float32),
        compiler_params=pltpu.CompilerParams(use_tc_tiling_on_sc=False),
        scratch_types=[
            pltpu.VMEM((n_chunks, CHUNK), jnp.int32),
            pltpu.VMEM((NBUF, CHUNK, D_MODEL), jnp.float32),
        ]
        + [pltpu.SemaphoreType.DMA] * (2 * NBUF),
    )
    def embed(tokens_hbm, table_hbm, out_hbm, idx_v, rows_v, *sems):
        gsem = sems[:NBUF]
        osem = sems[NBUF:]
        wid = lax.axis_index("s") * NUM_CORES + lax.axis_index("c")
        base = wid * b_per_w
        pltpu.sync_copy(tokens_hbm.at[wid], idx_v)

        def gather(j):
            b = j % NBUF
            pltpu.async_copy(table_hbm.at[idx_v.at[j]], rows_v.at[b], gsem[b])

        def wait_gather(j, b):
            pltpu.make_async_copy(
                table_hbm.at[idx_v.at[j]], rows_v.at[b], gsem[b]
            ).wait()

        def out_start(j, b):
            pltpu.async_copy(
                rows_v.at[b], out_hbm.at[pl.ds(base + j * CHUNK, CHUNK)], osem[b]
            )

        def wait_out(j, b):
            pltpu.make_async_copy(
                rows_v.at[b], out_hbm.at[pl.ds(base + j * CHUNK, CHUNK)], osem[b]
            ).wait()

        for j in range(PF):
            gather(j)

        for i in range(NBUF - PF):
            gather(i + PF)
            wait_gather(i, i % NBUF)
            out_start(i, i % NBUF)

        head = NBUF - PF
        n_main = n_chunks - PF - head

        def group(g, carry):
            for b in range(NBUF):
                i = head + g * NBUF + b
                bi = (head + b) % NBUF
                bpf = (head + b + PF) % NBUF
                wait_out(i - head, bpf)
                pltpu.async_copy(
                    table_hbm.at[idx_v.at[i + PF]], rows_v.at[bpf], gsem[bpf]
                )
                wait_gather(i, bi)
                out_start(i, bi)
            return carry

        lax.fori_loop(0, n_main // NBUF, group, 0)

        for i in range(n_chunks - PF, n_chunks):
            wait_gather(i, i % NBUF)
            out_start(i, i % NBUF)

        for j in range(n_chunks - NBUF, n_chunks):
            wait_out(j, j % NBUF)

    return embed


def kernel(tokens, W_E):
    b, s = tokens.shape
    n_tok = b * s
    flat = tokens.reshape(NW, n_tok // NW // CHUNK, CHUNK).astype(jnp.int32)
    out = _make_embed(n_tok)(flat, W_E)
    return out.reshape(b, s, D_MODEL)


# speedup vs baseline: 1.0022x; 1.0022x over previous
import functools

import jax
import jax.numpy as jnp
from jax import lax
from jax.experimental import pallas as pl
from jax.experimental.pallas import tpu as pltpu
from jax.experimental.pallas import tpu_sc as plsc

D_MODEL = 64
NUM_CORES = 2
NUM_SUBCORES = 16
NW = NUM_CORES * NUM_SUBCORES


def _make_embed(n_rows: int, seq: int):
    rows_per_w = n_rows // NW
    NBUF = 4
    PF = 3
    n_chunks = rows_per_w

    mesh = plsc.VectorSubcoreMesh(core_axis_name="c", subcore_axis_name="s")

    @functools.partial(
        pl.kernel,
        mesh=mesh,
        out_type=jax.ShapeDtypeStruct((n_rows, seq, D_MODEL), jnp.float32),
        compiler_params=pltpu.CompilerParams(use_tc_tiling_on_sc=False),
        scratch_types=[
            pltpu.VMEM((rows_per_w, seq), jnp.int32),
            pltpu.VMEM((NBUF, seq, D_MODEL), jnp.float32),
        ]
        + [pltpu.SemaphoreType.DMA] * (2 * NBUF),
    )
    def embed(tokens_hbm, table_hbm, out_hbm, idx_v, rows_v, *sems):
        gsem = sems[:NBUF]
        osem = sems[NBUF:]
        wid = lax.axis_index("s") * NUM_CORES + lax.axis_index("c")
        base = wid * rows_per_w
        pltpu.sync_copy(tokens_hbm.at[pl.ds(base, rows_per_w)], idx_v)

        def gather(j):
            b = j % NBUF
            pltpu.async_copy(table_hbm.at[idx_v.at[j]], rows_v.at[b], gsem[b])

        def wait_gather(j, b):
            pltpu.make_async_copy(
                table_hbm.at[idx_v.at[j]], rows_v.at[b], gsem[b]
            ).wait()

        def out_start(j, b):
            pltpu.async_copy(rows_v.at[b], out_hbm.at[base + j], osem[b])

        def wait_out(j, b):
            pltpu.make_async_copy(
                rows_v.at[b], out_hbm.at[base + j], osem[b]
            ).wait()

        for j in range(PF):
            gather(j)

        head = NBUF - PF
        for i in range(head):
            gather(i + PF)
            wait_gather(i, i % NBUF)
            out_start(i, i % NBUF)

        n_main = n_chunks - PF - head

        def group(g, carry):
            for b in range(NBUF):
                i = head + g * NBUF + b
                bi = (head + b) % NBUF
                bpf = (head + b + PF) % NBUF
                wait_out(i - head, bpf)
                pltpu.async_copy(
                    table_hbm.at[idx_v.at[i + PF]], rows_v.at[bpf], gsem[bpf]
                )
                wait_gather(i, bi)
                out_start(i, bi)
            return carry

        lax.fori_loop(0, n_main // NBUF, group, 0)

        for i in range(n_chunks - PF, n_chunks):
            wait_gather(i, i % NBUF)
            out_start(i, i % NBUF)

        for j in range(n_chunks - NBUF, n_chunks):
            wait_out(j, j % NBUF)

    return embed


def kernel(tokens, W_E):
    n_rows, seq = tokens.shape
    return _make_embed(n_rows, seq)(tokens.astype(jnp.int32), W_E)
